# baseline (speedup 1.0000x reference)
import functools

import jax
import jax.numpy as jnp
from jax import lax
from jax.experimental import pallas as pl
from jax.experimental.pallas import tpu as pltpu, tpu_sc as plsc

VOCAB = 1000000
DIM = 64
NB, NTOK = 4096, 200

NC, NS = 2, 16
NW = NC * NS

NBLK = 8
BW = NB // NBLK
NSG = NW // NBLK
SW = NTOK // NSG
SEG = 128
NQ = BW // SEG
RTP = BW + 8

TB = 512
NSTR = VOCAB // TB
TAIL = VOCAB - NSTR * TB
NIT = 62
TINP = TB + 1

_SC_PARAMS = pltpu.CompilerParams(
    use_tc_tiling_on_sc=False, needs_layout_passes=False
)


def _tbody(tt_hbm, tlin_hbm, tin0, tin1, tout, isem0, isem1):
    w = lax.axis_index("s") * NC + lax.axis_index("c")
    tin = (tin0, tin1)
    isem = (isem0, isem1)
    iota16 = lax.iota(jnp.int32, 16)

    def stripe(i):
        return jnp.minimum(w + NW * i, (NSTR - NW) + w) * TB

    def fire(i, b):
        pltpu.async_copy(
            tt_hbm.at[:, pl.ds(stripe(i), TB)],
            tin[b].at[:, pl.ds(0, TB)],
            isem[b],
        )

    def drain(i, b):
        pltpu.make_async_copy(
            tt_hbm.at[:, pl.ds(stripe(i), TB)],
            tin[b].at[:, pl.ds(0, TB)],
            isem[b],
        ).wait()

    def transpose(src, n):
        @plsc.parallel_loop(0, n, step=1, unroll=8)
        def _(j):
            jv = jnp.full((16,), j, jnp.int32)
            for k in range(DIM // 16):
                v = plsc.load_gather(src, [iota16 + 16 * k, jv])
                tout[j, pl.ds(16 * k, 16)] = v

    fire(0, 0)

    @pl.loop(0, NIT, step=2)
    def _(i0):
        for b in range(2):
            i = i0 + b
            drain(i, b)

            @pl.when(i + 1 < NIT)
            def _():
                fire(i + 1, 1 - b)

            transpose(tin[b], TB)
            pltpu.sync_copy(tout, tlin_hbm.at[pl.ds(stripe(i), TB)])

    @pl.when(w == 1)
    def _():
        pltpu.sync_copy(
            tt_hbm.at[:, pl.ds(NSTR * TB, TAIL)], tin0.at[:, pl.ds(0, TAIL)]
        )
        transpose(tin0, TAIL)
        pltpu.sync_copy(
            tout.at[pl.ds(0, TAIL)], tlin_hbm.at[pl.ds(NSTR * TB, TAIL)]
        )


def _gbody(xt_hbm, table_hbm, out_hbm, xv, rows0, rows1, rt, gsem0, gsem1):
    w = lax.axis_index("s") * NC + lax.axis_index("c")
    b0 = (w % NBLK) * BW
    s_base = (w // NBLK) * SW

    for q in range(NQ):
        pltpu.sync_copy(
            xt_hbm.at[pl.ds(s_base, SW), pl.ds(b0 + SEG * q, SEG)],
            xv.at[pl.ds(SW * q, SW)],
        )

    rows = (rows0, rows1)
    gsem = (gsem0, gsem1)
    iota16 = lax.iota(jnp.int32, 16)

    def fire(si, buf):
        for q in range(NQ):
            pltpu.async_copy(
                table_hbm.at[xv.at[SW * q + si]],
                rows[buf].at[pl.ds(SEG * q, SEG)],
                gsem[buf],
            )

    def drain(si, buf):
        for q in range(NQ):
            pltpu.make_async_copy(
                table_hbm.at[xv.at[SW * q + si]],
                rows[buf].at[pl.ds(SEG * q, SEG)],
                gsem[buf],
            ).wait()

    def transpose(buf):
        @plsc.parallel_loop(0, BW, step=1, unroll=8)
        def _(j):
            jv = jnp.full((16,), j, jnp.int32)
            for k2 in range(DIM // 16):
                v = rows[buf][j, pl.ds(16 * k2, 16)]
                plsc.store_scatter(rt, [iota16 + 16 * k2, jv], v)

    fire(0, 0)

    @pl.loop(0, SW, step=2)
    def _(s0):
        for b in range(2):
            si = s0 + b
            drain(si, b)

            @pl.when(si + 1 < SW)
            def _():
                fire(si + 1, 1 - b)

            transpose(b)
            pltpu.sync_copy(
                rt.at[:, pl.ds(0, BW)],
                out_hbm.at[s_base + si, :, pl.ds(b0, BW)],
            )


@jax.jit
def _lookup(x_t, table_t):
    mesh = plsc.VectorSubcoreMesh(core_axis_name="c", subcore_axis_name="s")
    tk = pl.kernel(
        _tbody,
        out_type=jax.ShapeDtypeStruct((VOCAB, DIM), jnp.float32),
        mesh=mesh,
        scratch_types=[
            pltpu.VMEM((DIM, TINP), jnp.float32),
            pltpu.VMEM((DIM, TINP), jnp.float32),
            pltpu.VMEM((TB, DIM), jnp.float32),
            pltpu.SemaphoreType.DMA,
            pltpu.SemaphoreType.DMA,
        ],
        compiler_params=_SC_PARAMS,
    )
    table_lin = tk(table_t)

    gk = pl.kernel(
        _gbody,
        out_type=jax.ShapeDtypeStruct((NTOK, DIM, NB), jnp.float32),
        mesh=mesh,
        scratch_types=[
            pltpu.VMEM((NTOK, SEG), jnp.int32),
            pltpu.VMEM((BW, DIM), jnp.float32),
            pltpu.VMEM((BW, DIM), jnp.float32),
            pltpu.VMEM((DIM, RTP), jnp.float32),
            pltpu.SemaphoreType.DMA,
            pltpu.SemaphoreType.DMA,
        ],
        compiler_params=_SC_PARAMS,
    )
    return gk(x_t, table_lin)


def kernel(x, table):
    x_t = jnp.transpose(x).astype(jnp.int32)
    table_t = jnp.transpose(table)
    out_t = _lookup(x_t, table_t)
    return jnp.transpose(out_t, (2, 0, 1))


# speedup vs baseline: 5.7904x; 5.7904x over previous
import functools

import jax
import jax.numpy as jnp
from jax import lax
from jax.experimental import pallas as pl
from jax.experimental.pallas import tpu as pltpu, tpu_sc as plsc

VOCAB = 1000000
DIM = 64
NB, NTOK = 4096, 200

NC, NS = 2, 16
NW = NC * NS
NBLK = 8
BW = NB // NBLK
NSG = NW // NBLK
SW = NTOK // NSG
SEG = 128
NQ = BW // SEG
RTP = BW + 8


def _body(xt_hbm, table_hbm, out_hbm, xv, rows0, rows1, rt, gsem0, gsem1):
    w = lax.axis_index("s") * NC + lax.axis_index("c")
    b0 = (w % NBLK) * BW
    s_base = (w // NBLK) * SW

    for q in range(NQ):
        pltpu.sync_copy(
            xt_hbm.at[pl.ds(s_base, SW), pl.ds(b0 + SEG * q, SEG)],
            xv.at[pl.ds(SW * q, SW)],
        )

    rows = (rows0, rows1)
    gsem = (gsem0, gsem1)

    iota16 = lax.iota(jnp.int32, 16)

    def fire(si, buf):
        for q in range(NQ):
            pltpu.async_copy(
                table_hbm.at[xv.at[SW * q + si]],
                rows[buf].at[pl.ds(SEG * q, SEG)],
                gsem[buf],
            )

    def drain(si, buf):
        for q in range(NQ):
            pltpu.make_async_copy(
                table_hbm.at[xv.at[SW * q + si]],
                rows[buf].at[pl.ds(SEG * q, SEG)],
                gsem[buf],
            ).wait()

    def transpose(buf):
        @plsc.parallel_loop(0, BW, step=1, unroll=8)
        def _(j):
            jv = jnp.full((16,), j, jnp.int32)
            for k2 in range(DIM // 16):
                v = rows[buf][j, pl.ds(16 * k2, 16)]
                plsc.store_scatter(rt, [iota16 + 16 * k2, jv], v)

    fire(0, 0)

    @pl.loop(0, SW, step=2)
    def _(s0):
        for b in range(2):
            si = s0 + b
            drain(si, b)

            @pl.when(si + 1 < SW)
            def _():
                fire(si + 1, 1 - b)

            transpose(b)
            pltpu.sync_copy(
                rt.at[:, pl.ds(0, BW)],
                out_hbm.at[pl.ds((s_base + si) * DIM, DIM), pl.ds(b0, BW)],
            )


@jax.jit
def _lookup(x_t, table_lin):
    mesh = plsc.VectorSubcoreMesh(core_axis_name="c", subcore_axis_name="s")
    k = pl.kernel(
        _body,
        out_type=jax.ShapeDtypeStruct((NTOK * DIM, NB), jnp.float32),
        mesh=mesh,
        scratch_types=[
            pltpu.VMEM((NTOK, SEG), jnp.int32),
            pltpu.VMEM((BW, DIM), jnp.float32),
            pltpu.VMEM((BW, DIM), jnp.float32),
            pltpu.VMEM((DIM, RTP), jnp.float32),
            pltpu.SemaphoreType.DMA,
            pltpu.SemaphoreType.DMA,
        ],
        compiler_params=pltpu.CompilerParams(
            use_tc_tiling_on_sc=False, needs_layout_passes=False
        ),
    )
    return k(x_t, table_lin)


def kernel(x, table):
    x_t = jnp.transpose(x).astype(jnp.int32)
    t_pair = jax.lax.optimization_barrier(jnp.reshape(table, (VOCAB // 2, 2 * DIM)))
    t_lin = jnp.reshape(t_pair, (VOCAB, DIM))
    out2 = _lookup(x_t, t_lin)
    out_t = jnp.reshape(out2, (NTOK, DIM, NB))
    return jnp.transpose(out_t, (2, 0, 1))
